# initial kernel scaffold (unmeasured)
import jax
import jax.numpy as jnp
from jax import lax
from jax.experimental import pallas as pl
from jax.experimental.pallas import tpu as pltpu

N_DEV = 4
B = 2
S_LOC = 256
S_GLB = N_DEV * S_LOC
D_MODEL = 768
HQ = 4
DH = 64
HD = HQ * DH
SCALE = 0.125


def kernel(x, Wq, Wk, Wv, Wo):
    def body(x_ref, wq_ref, wk_ref, wv_ref, wo_ref, out_ref,
             kv_ref, send_sems, recv_sems):
        my = lax.axis_index("i")
        left = lax.rem(my + N_DEV - 1, N_DEV)
        right = lax.rem(my + 1, N_DEV)

        barrier_sem = pltpu.get_barrier_semaphore()
        for nbr in (left, right):
            pl.semaphore_signal(barrier_sem, inc=1, device_id=(nbr,),
                                device_id_type=pl.DeviceIdType.MESH)
        pl.semaphore_wait(barrier_sem, 2)

        pos = lax.broadcasted_iota(jnp.float32, (S_LOC, HD), 0) \
            + (my * S_LOC).astype(jnp.float32)
        col = lax.broadcasted_iota(jnp.int32, (S_LOC, HD), 1)
        d = col % DH
        expo = (2 * (d // 2)).astype(jnp.float32) / DH
        inv = jnp.exp(-jnp.log(10000.0) * expo)
        theta = pos * inv
        cos_t = jnp.cos(theta)
        sin_t = jnp.sin(theta)

        rr = lax.broadcasted_iota(jnp.int32, (HD, HD), 0)
        cc = lax.broadcasted_iota(jnp.int32, (HD, HD), 1)
        rot_mat = jnp.where((rr % 2 == 0) & (cc == rr + 1), 1.0, 0.0) \
                + jnp.where((rr % 2 == 1) & (cc == rr - 1), -1.0, 0.0)

        def rope(t):
            t_rot = lax.dot_general(t, rot_mat, (((1,), (0,)), ((), ())),
                                    preferred_element_type=jnp.float32)
            return t * cos_t + t_rot * sin_t

        qs = []
        for b in range(B):
            xb = x_ref[b]
            q = rope(lax.dot_general(xb, wq_ref[...], (((1,), (0,)), ((), ())),
                                     preferred_element_type=jnp.float32))
            k = rope(lax.dot_general(xb, wk_ref[...], (((1,), (0,)), ((), ())),
                                     preferred_element_type=jnp.float32))
            v = lax.dot_general(xb, wv_ref[...], (((1,), (0,)), ((), ())),
                                preferred_element_type=jnp.float32)
            qs.append(q)
            kv_ref[b, pl.ds(my * S_LOC, S_LOC), :] = k
            kv_ref[B + b, pl.ds(my * S_LOC, S_LOC), :] = v

        for h in range(N_DEV - 1):
            origin = lax.rem(my - h + N_DEV, N_DEV)
            seg = pl.ds(origin * S_LOC, S_LOC)
            rdma = pltpu.make_async_remote_copy(
                src_ref=kv_ref.at[:, seg, :],
                dst_ref=kv_ref.at[:, seg, :],
                send_sem=send_sems.at[h],
                recv_sem=recv_sems.at[h],
                device_id=(right,),
                device_id_type=pl.DeviceIdType.MESH,
            )
            rdma.start()
            rdma.wait()

        for b in range(B):
            q = qs[b]
            outs = []
            for hh in range(HQ):
                qh = q[:, hh * DH:(hh + 1) * DH]
                kh = kv_ref[b, :, hh * DH:(hh + 1) * DH]
                s = lax.dot_general(qh, kh, (((1,), (1,)), ((), ())),
                                    preferred_element_type=jnp.float32) * SCALE
                m = jnp.max(s, axis=1, keepdims=True)
                w = jnp.exp(s - m)
                w = w / jnp.sum(w, axis=1, keepdims=True)
                vh = kv_ref[B + b, :, hh * DH:(hh + 1) * DH]
                outs.append(lax.dot_general(w, vh, (((1,), (0,)), ((), ())),
                                            preferred_element_type=jnp.float32))
            ctx = jnp.concatenate(outs, axis=1)
            out_ref[b] = lax.dot_general(ctx, wo_ref[...],
                                         (((1,), (0,)), ((), ())),
                                         preferred_element_type=jnp.float32)

    return pl.pallas_call(
        body,
        out_shape=jax.ShapeDtypeStruct((B, S_LOC, D_MODEL), jnp.float32),
        in_specs=[pl.BlockSpec(memory_space=pltpu.VMEM)] * 5,
        out_specs=pl.BlockSpec(memory_space=pltpu.VMEM),
        scratch_shapes=[
            pltpu.VMEM((2 * B, S_GLB, HD), jnp.float32),
            pltpu.SemaphoreType.DMA((N_DEV - 1,)),
            pltpu.SemaphoreType.DMA((N_DEV - 1,)),
        ],
        compiler_params=pltpu.CompilerParams(collective_id=0),
    )(x, Wq, Wk, Wv, Wo)


# baseline (device time: 57918 ns/iter reference)
import jax
import jax.numpy as jnp
from jax import lax
from jax.experimental import pallas as pl
from jax.experimental.pallas import tpu as pltpu

N_DEV = 4
B = 2
S_LOC = 256
S_GLB = N_DEV * S_LOC
D_MODEL = 768
HQ = 4
DH = 64
HD = HQ * DH
SCALE = 0.125


def kernel(x, Wq, Wk, Wv, Wo):
    def body(x_ref, wq_ref, wk_ref, wv_ref, wo_ref, out_ref,
             kv_ref, send_sems, recv_sems):
        my = lax.axis_index("i")
        left = lax.rem(my + N_DEV - 1, N_DEV)
        right = lax.rem(my + 1, N_DEV)

        barrier_sem = pltpu.get_barrier_semaphore()
        for nbr in (left, right):
            pl.semaphore_signal(barrier_sem, inc=1, device_id=(nbr,),
                                device_id_type=pl.DeviceIdType.MESH)
        pl.semaphore_wait(barrier_sem, 2)

        pos = (lax.broadcasted_iota(jnp.int32, (S_LOC, HD), 0)
               + my * S_LOC).astype(jnp.float32)
        col = lax.broadcasted_iota(jnp.int32, (S_LOC, HD), 1)
        d = col % DH
        expo = (2 * (d // 2)).astype(jnp.float32) / DH
        inv = jnp.exp(-jnp.log(10000.0) * expo)
        theta = pos * inv
        cos_t = jnp.cos(theta)
        sin_t = jnp.sin(theta)

        rr = lax.broadcasted_iota(jnp.int32, (HD, HD), 0)
        cc = lax.broadcasted_iota(jnp.int32, (HD, HD), 1)
        rot_mat = jnp.where((rr % 2 == 0) & (cc == rr + 1), 1.0, 0.0) \
                + jnp.where((rr % 2 == 1) & (cc == rr - 1), -1.0, 0.0)

        def rope(t):
            t_rot = lax.dot_general(t, rot_mat, (((1,), (0,)), ((), ())),
                                    preferred_element_type=jnp.float32)
            return t * cos_t + t_rot * sin_t

        qs = []
        for b in range(B):
            xb = x_ref[b]
            q = rope(lax.dot_general(xb, wq_ref[...], (((1,), (0,)), ((), ())),
                                     preferred_element_type=jnp.float32))
            k = rope(lax.dot_general(xb, wk_ref[...], (((1,), (0,)), ((), ())),
                                     preferred_element_type=jnp.float32))
            v = lax.dot_general(xb, wv_ref[...], (((1,), (0,)), ((), ())),
                                preferred_element_type=jnp.float32)
            qs.append(q)
            kv_ref[b, pl.ds(my * S_LOC, S_LOC), :] = k
            kv_ref[B + b, pl.ds(my * S_LOC, S_LOC), :] = v

        for h in range(N_DEV - 1):
            origin = lax.rem(my - h + N_DEV, N_DEV)
            seg = pl.ds(origin * S_LOC, S_LOC)
            rdma = pltpu.make_async_remote_copy(
                src_ref=kv_ref.at[:, seg, :],
                dst_ref=kv_ref.at[:, seg, :],
                send_sem=send_sems.at[h],
                recv_sem=recv_sems.at[h],
                device_id=(right,),
                device_id_type=pl.DeviceIdType.MESH,
            )
            rdma.start()
            rdma.wait()

        for b in range(B):
            q = qs[b]
            outs = []
            for hh in range(HQ):
                qh = q[:, hh * DH:(hh + 1) * DH]
                kh = kv_ref[b, :, hh * DH:(hh + 1) * DH]
                s = lax.dot_general(qh, kh, (((1,), (1,)), ((), ())),
                                    preferred_element_type=jnp.float32) * SCALE
                m = jnp.max(s, axis=1, keepdims=True)
                w = jnp.exp(s - m)
                w = w / jnp.sum(w, axis=1, keepdims=True)
                vh = kv_ref[B + b, :, hh * DH:(hh + 1) * DH]
                outs.append(lax.dot_general(w, vh, (((1,), (0,)), ((), ())),
                                            preferred_element_type=jnp.float32))
            ctx = jnp.concatenate(outs, axis=1)
            out_ref[b] = lax.dot_general(ctx, wo_ref[...],
                                         (((1,), (0,)), ((), ())),
                                         preferred_element_type=jnp.float32)

    return pl.pallas_call(
        body,
        out_shape=jax.ShapeDtypeStruct((B, S_LOC, D_MODEL), jnp.float32),
        in_specs=[pl.BlockSpec(memory_space=pltpu.VMEM)] * 5,
        out_specs=pl.BlockSpec(memory_space=pltpu.VMEM),
        scratch_shapes=[
            pltpu.VMEM((2 * B, S_GLB, HD), jnp.float32),
            pltpu.SemaphoreType.DMA((N_DEV - 1,)),
            pltpu.SemaphoreType.DMA((N_DEV - 1,)),
        ],
        compiler_params=pltpu.CompilerParams(collective_id=0),
    )(x, Wq, Wk, Wv, Wo)


# device time: 36463 ns/iter; 1.5884x vs baseline; 1.5884x over previous
import jax
import jax.numpy as jnp
from jax import lax
from jax.experimental import pallas as pl
from jax.experimental.pallas import tpu as pltpu

N_DEV = 4
B = 2
S_LOC = 256
S_GLB = N_DEV * S_LOC
D_MODEL = 768
HQ = 4
DH = 64
HD = HQ * DH
SCALE = 0.125
NEG_BIG = -1e30


def kernel(x, Wq, Wk, Wv, Wo):
    def body(x_ref, wq_ref, wk_ref, wv_ref, wo_ref, out_ref,
             kv_ref, send_sems, recv_sems):
        my = lax.axis_index("i")
        left = lax.rem(my + N_DEV - 1, N_DEV)
        right = lax.rem(my + 1, N_DEV)
        diag = lax.rem(my + 2, N_DEV)

        barrier_sem = pltpu.get_barrier_semaphore()
        for nbr in (left, right):
            pl.semaphore_signal(barrier_sem, inc=1, device_id=(nbr,),
                                device_id_type=pl.DeviceIdType.MESH)
        pl.semaphore_wait(barrier_sem, 2)

        pos = (lax.broadcasted_iota(jnp.int32, (S_LOC, HD), 0)
               + my * S_LOC).astype(jnp.float32)
        col = lax.broadcasted_iota(jnp.int32, (S_LOC, HD), 1)
        d = col % DH
        expo = (2 * (d // 2)).astype(jnp.float32) / DH
        inv = jnp.exp(-jnp.log(10000.0) * expo)
        theta = pos * inv
        cos_t = jnp.cos(theta)
        sin_t = jnp.sin(theta)

        rr = lax.broadcasted_iota(jnp.int32, (HD, HD), 0)
        cc = lax.broadcasted_iota(jnp.int32, (HD, HD), 1)
        rot_mat = jnp.where((rr % 2 == 0) & (cc == rr + 1), 1.0, 0.0) \
                + jnp.where((rr % 2 == 1) & (cc == rr - 1), -1.0, 0.0)

        def mm(a, b_mat):
            return lax.dot_general(a, b_mat, (((1,), (0,)), ((), ())),
                                   preferred_element_type=jnp.float32)

        def rope(t):
            return t * cos_t + mm(t, rot_mat) * sin_t

        for b in range(B):
            xb = x_ref[b]
            kv_ref[b, pl.ds(my * S_LOC, S_LOC), :] = rope(mm(xb, wk_ref[...]))
            kv_ref[B + b, pl.ds(my * S_LOC, S_LOC), :] = mm(xb, wv_ref[...])

        def seg(origin):
            return pl.ds(origin * S_LOC, S_LOC)

        dA = pltpu.make_async_remote_copy(
            src_ref=kv_ref.at[:, seg(my), :],
            dst_ref=kv_ref.at[:, seg(my), :],
            send_sem=send_sems.at[0], recv_sem=recv_sems.at[0],
            device_id=(right,), device_id_type=pl.DeviceIdType.MESH)
        dB = pltpu.make_async_remote_copy(
            src_ref=kv_ref.at[:, seg(my), :],
            dst_ref=kv_ref.at[:, seg(my), :],
            send_sem=send_sems.at[1], recv_sem=recv_sems.at[1],
            device_id=(left,), device_id_type=pl.DeviceIdType.MESH)
        dA.start()
        dB.start()

        qs = [rope(mm(x_ref[b], wq_ref[...])) for b in range(B)]

        m0 = jnp.full((S_LOC, 1), NEG_BIG, jnp.float32)
        l0 = jnp.zeros((S_LOC, 1), jnp.float32)
        a0 = jnp.zeros((S_LOC, DH), jnp.float32)
        states = [[(m0, l0, a0) for _ in range(HQ)] for _ in range(B)]

        def process(origin):
            for b in range(B):
                for hh in range(HQ):
                    m, l, acc = states[b][hh]
                    cols = slice(hh * DH, (hh + 1) * DH)
                    qh = qs[b][:, cols]
                    kh = kv_ref[b, seg(origin), cols]
                    vh = kv_ref[B + b, seg(origin), cols]
                    s = lax.dot_general(qh, kh, (((1,), (1,)), ((), ())),
                                        preferred_element_type=jnp.float32)
                    s = s * SCALE
                    m_new = jnp.maximum(m, jnp.max(s, axis=1, keepdims=True))
                    alpha = jnp.exp(m - m_new)
                    p = jnp.exp(s - m_new)
                    l = l * alpha + jnp.sum(p, axis=1, keepdims=True)
                    acc = acc * alpha + mm(p, vh)
                    states[b][hh] = (m_new, l, acc)

        process(my)

        dA.wait_recv()
        dC = pltpu.make_async_remote_copy(
            src_ref=kv_ref.at[pl.ds(0, B), seg(left), :],
            dst_ref=kv_ref.at[pl.ds(0, B), seg(left), :],
            send_sem=send_sems.at[2], recv_sem=recv_sems.at[2],
            device_id=(right,), device_id_type=pl.DeviceIdType.MESH)
        dC.start()
        process(left)

        dB.wait_recv()
        dD = pltpu.make_async_remote_copy(
            src_ref=kv_ref.at[pl.ds(B, B), seg(right), :],
            dst_ref=kv_ref.at[pl.ds(B, B), seg(right), :],
            send_sem=send_sems.at[3], recv_sem=recv_sems.at[3],
            device_id=(left,), device_id_type=pl.DeviceIdType.MESH)
        dD.start()
        process(right)

        dC.wait_recv()
        dD.wait_recv()
        process(diag)

        for b in range(B):
            ctx = jnp.concatenate(
                [acc / l for (m, l, acc) in states[b]], axis=1)
            out_ref[b] = mm(ctx, wo_ref[...])

        dA.wait_send()
        dB.wait_send()
        dC.wait_send()
        dD.wait_send()

    return pl.pallas_call(
        body,
        out_shape=jax.ShapeDtypeStruct((B, S_LOC, D_MODEL), jnp.float32),
        in_specs=[pl.BlockSpec(memory_space=pltpu.VMEM)] * 5,
        out_specs=pl.BlockSpec(memory_space=pltpu.VMEM),
        scratch_shapes=[
            pltpu.VMEM((2 * B, S_GLB, HD), jnp.float32),
            pltpu.SemaphoreType.DMA((4,)),
            pltpu.SemaphoreType.DMA((4,)),
        ],
        compiler_params=pltpu.CompilerParams(collective_id=0),
    )(x, Wq, Wk, Wv, Wo)
